# baseline (speedup 1.0000x reference)
import jax
import jax.numpy as jnp
from jax import lax
from jax.experimental import pallas as pl
from jax.experimental.pallas import tpu as pltpu
from jax.experimental.pallas import tpu_sc as plsc

NUM_LABELS = 1000
NUM_WEIGHTS = 256
EMB_DIM = 64
BATCH = 4096
IMG_DIM = 1024
OUT_DIM = EMB_DIM + IMG_DIM
TOTAL_ROWS = NUM_WEIGHTS + BATCH
LANE = 128

_SC_INFO = plsc.get_sparse_core_info()
_NW = _SC_INFO.num_cores * _SC_INFO.num_subcores
_B_PER_W = BATCH // _NW

CHUNK = 1024
NCHUNK = BATCH // CHUNK


def _sc_gather_body(table_hbm, idx_hbm, out_hbm, idx_v, rows_v, sem):
    wid = lax.axis_index("s") * _SC_INFO.num_cores + lax.axis_index("c")
    base = wid * _B_PER_W
    pltpu.sync_copy(idx_hbm.at[pl.ds(base, _B_PER_W)], idx_v)
    pltpu.async_copy(table_hbm.at[idx_v], rows_v, sem).wait()
    pltpu.sync_copy(rows_v, out_hbm.at[pl.ds(base, _B_PER_W)])


def _sc_gather(table128, labels):
    mesh = plsc.VectorSubcoreMesh(core_axis_name="c", subcore_axis_name="s")
    return pl.kernel(
        _sc_gather_body,
        mesh=mesh,
        out_type=jax.ShapeDtypeStruct((BATCH, LANE), jnp.float32),
        scratch_types=[
            pltpu.VMEM((_B_PER_W,), jnp.int32),
            pltpu.VMEM((_B_PER_W, LANE), jnp.float32),
            pltpu.SemaphoreType.DMA,
        ],
    )(table128, labels)


def _tc_body(enc_hbm, img_hbm, w_hbm, out_hbm,
             ib0, ib1, lb0, lb1, ob0, ob1, tb, wv,
             isem0, isem1, lsem0, lsem1, osem0, osem1, tsem, wsem):
    ibufs, lbufs, obufs = [ib0, ib1], [lb0, lb1], [ob0, ob1]
    isems, lsems, osems = [isem0, isem1], [lsem0, lsem1], [osem0, osem1]

    def start_in(i, sl):
        ic = pltpu.make_async_copy(
            img_hbm.at[pl.ds(i * CHUNK, CHUNK)], ibufs[sl], isems[sl])
        lc = pltpu.make_async_copy(
            enc_hbm.at[pl.ds(i * CHUNK, CHUNK)], lbufs[sl], lsems[sl])
        ic.start()
        lc.start()
        return (ic, lc)

    wcp = pltpu.make_async_copy(w_hbm, wv, wsem)
    wcp.start()

    started_in = [start_in(0, 0), start_in(1, 1)]

    wcp.wait()
    tb[...] = jnp.concatenate(
        [wv[...], jnp.zeros((NUM_WEIGHTS, IMG_DIM), jnp.float32)], axis=1)
    tcp = pltpu.make_async_copy(tb, out_hbm.at[pl.ds(0, NUM_WEIGHTS)], tsem)
    tcp.start()

    started_out = {}
    for i in range(NCHUNK):
        sl = i % 2
        for c in started_in[i]:
            c.wait()
        if i >= 2:
            started_out[i - 2].wait()
        obufs[sl][...] = jnp.concatenate(
            [lbufs[sl][:, :EMB_DIM], ibufs[sl][...]], axis=1)
        oc = pltpu.make_async_copy(
            obufs[sl],
            out_hbm.at[pl.ds(NUM_WEIGHTS + i * CHUNK, CHUNK)], osems[sl])
        oc.start()
        started_out[i] = oc
        if i + 2 < NCHUNK:
            started_in.append(start_in(i + 2, sl))

    started_out[NCHUNK - 2].wait()
    started_out[NCHUNK - 1].wait()
    tcp.wait()


@jax.jit
def kernel(images, labels, label_embs, weight_embs):
    table128 = jnp.zeros((NUM_LABELS + 1, LANE), jnp.float32)
    table128 = lax.dynamic_update_slice(table128, label_embs, (0, 0))
    encoded = _sc_gather(table128, labels)

    out = pl.pallas_call(
        _tc_body,
        in_specs=[
            pl.BlockSpec(memory_space=pl.ANY),
            pl.BlockSpec(memory_space=pl.ANY),
            pl.BlockSpec(memory_space=pl.ANY),
        ],
        out_specs=pl.BlockSpec(memory_space=pl.ANY),
        out_shape=jax.ShapeDtypeStruct((TOTAL_ROWS, OUT_DIM), jnp.float32),
        scratch_shapes=[
            pltpu.VMEM((CHUNK, IMG_DIM), jnp.float32),
            pltpu.VMEM((CHUNK, IMG_DIM), jnp.float32),
            pltpu.VMEM((CHUNK, LANE), jnp.float32),
            pltpu.VMEM((CHUNK, LANE), jnp.float32),
            pltpu.VMEM((CHUNK, OUT_DIM), jnp.float32),
            pltpu.VMEM((CHUNK, OUT_DIM), jnp.float32),
            pltpu.VMEM((NUM_WEIGHTS, OUT_DIM), jnp.float32),
            pltpu.VMEM((NUM_WEIGHTS, EMB_DIM), jnp.float32),
            pltpu.SemaphoreType.DMA,
            pltpu.SemaphoreType.DMA,
            pltpu.SemaphoreType.DMA,
            pltpu.SemaphoreType.DMA,
            pltpu.SemaphoreType.DMA,
            pltpu.SemaphoreType.DMA,
            pltpu.SemaphoreType.DMA,
            pltpu.SemaphoreType.DMA,
        ],
        compiler_params=pltpu.CompilerParams(
            vmem_limit_bytes=100 * 1024 * 1024,
        ),
    )(encoded, images, weight_embs)
    return out


# speedup vs baseline: 1.3342x; 1.3342x over previous
import jax
import jax.numpy as jnp
from jax.experimental import pallas as pl
from jax.experimental.pallas import tpu as pltpu

NUM_LABELS = 1000
NUM_WEIGHTS = 256
EMB_DIM = 64
BATCH = 4096
IMG_DIM = 1024
OUT_DIM = EMB_DIM + IMG_DIM
TOTAL_ROWS = NUM_WEIGHTS + BATCH
TABLE_PAD = NUM_LABELS + 1

CHUNK = 1024
NCHUNK = BATCH // CHUNK


def _tc_body(lbl_hbm, table_ref, w_hbm, img_hbm, out_hbm,
             ib0, ib1, ob0, ob1, tb, wv, lblv,
             isem0, isem1, osem0, osem1, tsem, wsem, lsem):
    ibufs, obufs = [ib0, ib1], [ob0, ob1]
    isems, osems = [isem0, isem1], [osem0, osem1]

    def start_in(i, sl):
        c = pltpu.make_async_copy(
            img_hbm.at[pl.ds(i * CHUNK, CHUNK)], ibufs[sl], isems[sl])
        c.start()
        return c

    wcp = pltpu.make_async_copy(w_hbm, wv, wsem)
    wcp.start()
    lcp = pltpu.make_async_copy(lbl_hbm, lblv, lsem)
    lcp.start()

    started_in = [start_in(0, 0), start_in(1, 1)]

    wcp.wait()
    tb[...] = jnp.concatenate(
        [wv[...], jnp.zeros((NUM_WEIGHTS, IMG_DIM), jnp.float32)], axis=1)
    tcp = pltpu.make_async_copy(tb, out_hbm.at[pl.ds(0, NUM_WEIGHTS)], tsem)
    tcp.start()
    lcp.wait()

    started_out = {}
    for i in range(NCHUNK):
        sl = i % 2
        started_in[i].wait()
        if i >= 2:
            started_out[i - 2].wait()
        lbl = lblv[pl.ds(i * CHUNK, CHUNK), :]
        iota = jax.lax.broadcasted_iota(jnp.int32, (CHUNK, TABLE_PAD), 1)
        onehot = (iota == lbl).astype(jnp.float32)
        enc = jax.lax.dot_general(
            onehot, table_ref[...],
            dimension_numbers=(((1,), (0,)), ((), ())),
            preferred_element_type=jnp.float32,
        )
        obufs[sl][...] = jnp.concatenate([enc, ibufs[sl][...]], axis=1)
        oc = pltpu.make_async_copy(
            obufs[sl],
            out_hbm.at[pl.ds(NUM_WEIGHTS + i * CHUNK, CHUNK)], osems[sl])
        oc.start()
        started_out[i] = oc
        if i + 2 < NCHUNK:
            started_in.append(start_in(i + 2, sl))

    started_out[NCHUNK - 2].wait()
    started_out[NCHUNK - 1].wait()
    tcp.wait()


@jax.jit
def kernel(images, labels, label_embs, weight_embs):
    lbl2d = labels.reshape(BATCH, 1)

    out = pl.pallas_call(
        _tc_body,
        in_specs=[
            pl.BlockSpec(memory_space=pl.ANY),
            pl.BlockSpec(memory_space=pltpu.VMEM),
            pl.BlockSpec(memory_space=pl.ANY),
            pl.BlockSpec(memory_space=pl.ANY),
        ],
        out_specs=pl.BlockSpec(memory_space=pl.ANY),
        out_shape=jax.ShapeDtypeStruct((TOTAL_ROWS, OUT_DIM), jnp.float32),
        scratch_shapes=[
            pltpu.VMEM((CHUNK, IMG_DIM), jnp.float32),
            pltpu.VMEM((CHUNK, IMG_DIM), jnp.float32),
            pltpu.VMEM((CHUNK, OUT_DIM), jnp.float32),
            pltpu.VMEM((CHUNK, OUT_DIM), jnp.float32),
            pltpu.VMEM((NUM_WEIGHTS, OUT_DIM), jnp.float32),
            pltpu.VMEM((NUM_WEIGHTS, EMB_DIM), jnp.float32),
            pltpu.VMEM((BATCH, 1), jnp.int32),
            pltpu.SemaphoreType.DMA,
            pltpu.SemaphoreType.DMA,
            pltpu.SemaphoreType.DMA,
            pltpu.SemaphoreType.DMA,
            pltpu.SemaphoreType.DMA,
            pltpu.SemaphoreType.DMA,
            pltpu.SemaphoreType.DMA,
        ],
        compiler_params=pltpu.CompilerParams(
            vmem_limit_bytes=100 * 1024 * 1024,
        ),
    )(lbl2d, label_embs, weight_embs, images)
    return out
